# Initial kernel scaffold; baseline (speedup 1.0000x reference)
import jax
import jax.numpy as jnp
from jax.experimental import pallas as pl


def kernel(x, noise, W_enc, b_enc, W_dec, b_dec, mask_token):
    raise NotImplementedError("write your pallas kernel here")


# speedup vs baseline: 241.8772x; 241.8772x over previous
import jax
import jax.numpy as jnp
from jax.experimental import pallas as pl

_MASK_RATIO = 0.75


def _mae_body(x_ref, n_ref, we_ref, be_ref, wd_ref, bd_ref, mt_ref,
              dec_ref, mask_ref):
    NB, _, L = x_ref.shape
    KEEP = int(L * (1.0 - _MASK_RATIO))

    n = n_ref[...]
    nT = jnp.transpose(n)
    li = jax.lax.broadcasted_iota(jnp.int32, (L, L), 0)
    lj = jax.lax.broadcasted_iota(jnp.int32, (L, L), 1)

    we = we_ref[...]
    be = be_ref[...]
    wd = wd_ref[...]
    bd = bd_ref[...]
    const_dec = jnp.dot(mt_ref[...], wd, preferred_element_type=jnp.float32) + bd

    for b in range(NB):
        nb = n[b:b + 1, :]
        nbT = nT[:, b:b + 1]
        p = (nb < nbT) | ((nb == nbT) & (lj < li))
        rank_col = jnp.sum(p.astype(jnp.float32), axis=1, keepdims=True)
        keep_col = rank_col < KEEP
        q = (nbT < nb) | ((nbT == nb) & (li < lj))
        rank_row = jnp.sum(q.astype(jnp.float32), axis=0, keepdims=True)
        mask_ref[b:b + 1, :] = jnp.where(rank_row < KEEP, 0.0, 1.0)

        xb = x_ref[b]
        enc = jax.lax.dot_general(xb, we, (((0,), (0,)), ((), ())),
                                  preferred_element_type=jnp.float32) + be
        dec = jnp.dot(enc, wd, preferred_element_type=jnp.float32) + bd
        dec_ref[b] = jnp.where(keep_col, dec, const_dec)


def kernel(x, noise, W_enc, b_enc, W_dec, b_dec, mask_token):
    B, C, H, W = x.shape
    L = H * W
    d_enc = W_enc.shape[1]
    x3 = x.reshape(B, C, L)

    NB = 8
    grid = (B // NB,)

    dec, mask = pl.pallas_call(
        _mae_body,
        grid=grid,
        in_specs=[
            pl.BlockSpec((NB, C, L), lambda i: (i, 0, 0)),
            pl.BlockSpec((NB, L), lambda i: (i, 0)),
            pl.BlockSpec((C, d_enc), lambda i: (0, 0)),
            pl.BlockSpec((1, d_enc), lambda i: (0, 0)),
            pl.BlockSpec((d_enc, C), lambda i: (0, 0)),
            pl.BlockSpec((1, C), lambda i: (0, 0)),
            pl.BlockSpec((1, d_enc), lambda i: (0, 0)),
        ],
        out_specs=[
            pl.BlockSpec((NB, L, C), lambda i: (i, 0, 0)),
            pl.BlockSpec((NB, L), lambda i: (i, 0)),
        ],
        out_shape=[
            jax.ShapeDtypeStruct((B, L, C), jnp.float32),
            jax.ShapeDtypeStruct((B, L), jnp.float32),
        ],
    )(x3, noise, W_enc, b_enc.reshape(1, d_enc), W_dec,
      b_dec.reshape(1, C), mask_token.reshape(1, d_enc))
    return dec, mask


# speedup vs baseline: 263.2699x; 1.0884x over previous
import jax
import jax.numpy as jnp
from jax.experimental import pallas as pl
from jax.experimental.pallas import tpu as pltpu

_MASK_RATIO = 0.75


def _mae_body(x_ref, n_ref, we_ref, be_ref, wd_ref, bd_ref, mt_ref,
              dec_ref, mask_ref, maug_ref, const_ref):
    NB, C, L = x_ref.shape
    KEEP = int(L * (1.0 - _MASK_RATIO))
    KAUG = maug_ref.shape[0]

    @pl.when(pl.program_id(0) == 0)
    def _init():
        wd = wd_ref[...]
        bd = bd_ref[...]
        maug_ref[pl.ds(0, C), :] = jnp.dot(we_ref[...], wd,
                                           preferred_element_type=jnp.float32)
        c_row = jnp.dot(be_ref[...], wd, preferred_element_type=jnp.float32) + bd
        const_row = jnp.dot(mt_ref[...], wd, preferred_element_type=jnp.float32) + bd
        maug_ref[pl.ds(C, 8), :] = jnp.broadcast_to((c_row - const_row) * 0.125,
                                                    (8, C))
        const_ref[...] = const_row

    n = n_ref[...]
    nT = jnp.transpose(n)
    li = jax.lax.broadcasted_iota(jnp.int32, (L, L), 0)
    lj = jax.lax.broadcasted_iota(jnp.int32, (L, L), 1)
    idx_lt = li < lj
    maug = maug_ref[...]
    const_row = const_ref[...]

    for b in range(NB):
        nb = n[b:b + 1, :]
        nbT = nT[:, b:b + 1]
        q = (nbT < nb) | ((nbT == nb) & idx_lt)
        rank_row = jnp.sum(q.astype(jnp.float32), axis=0, keepdims=True)
        keep_row = jnp.where(rank_row < KEEP, 1.0, 0.0)
        mask_ref[b:b + 1, :] = 1.0 - keep_row

        x_aug = jnp.concatenate(
            [x_ref[b] * keep_row, jnp.broadcast_to(keep_row, (KAUG - C, L))],
            axis=0)
        dec = jax.lax.dot_general(x_aug, maug, (((0,), (0,)), ((), ())),
                                  preferred_element_type=jnp.float32)
        dec_ref[b] = dec + const_row


def kernel(x, noise, W_enc, b_enc, W_dec, b_dec, mask_token):
    B, C, H, W = x.shape
    L = H * W
    d_enc = W_enc.shape[1]
    x3 = x.reshape(B, C, L)

    NB = 8
    grid = (B // NB,)

    dec, mask = pl.pallas_call(
        _mae_body,
        grid=grid,
        in_specs=[
            pl.BlockSpec((NB, C, L), lambda i: (i, 0, 0)),
            pl.BlockSpec((NB, L), lambda i: (i, 0)),
            pl.BlockSpec((C, d_enc), lambda i: (0, 0)),
            pl.BlockSpec((1, d_enc), lambda i: (0, 0)),
            pl.BlockSpec((d_enc, C), lambda i: (0, 0)),
            pl.BlockSpec((1, C), lambda i: (0, 0)),
            pl.BlockSpec((1, d_enc), lambda i: (0, 0)),
        ],
        out_specs=[
            pl.BlockSpec((NB, L, C), lambda i: (i, 0, 0)),
            pl.BlockSpec((NB, L), lambda i: (i, 0)),
        ],
        out_shape=[
            jax.ShapeDtypeStruct((B, L, C), jnp.float32),
            jax.ShapeDtypeStruct((B, L), jnp.float32),
        ],
        scratch_shapes=[
            pltpu.VMEM((C + 8, C), jnp.float32),
            pltpu.VMEM((1, C), jnp.float32),
        ],
    )(x3, noise, W_enc, b_enc.reshape(1, d_enc), W_dec,
      b_dec.reshape(1, C), mask_token.reshape(1, d_enc))
    return dec, mask


# speedup vs baseline: 1023.1282x; 3.8862x over previous
import jax
import jax.numpy as jnp
from jax.experimental import pallas as pl
from jax.experimental.pallas import tpu as pltpu

_MASK_RATIO = 0.75


def _mae_body(x_ref, n_ref, we_ref, wdt_ref, be_ref, bd_ref, mt_ref,
              dec_ref, mask_ref, maug_ref, const_ref):
    LB, C, B = x_ref.shape
    L = n_ref.shape[0]
    KEEP = int(L * (1.0 - _MASK_RATIO))

    @pl.when(pl.program_id(0) == 0)
    def _init():
        wdt = wdt_ref[...]
        bd_col = bd_ref[...]
        m_t = jnp.dot(wdt, jnp.transpose(we_ref[...]),
                      preferred_element_type=jnp.float32)
        c_col = jnp.dot(wdt, be_ref[...],
                        preferred_element_type=jnp.float32) + bd_col
        const_col = jnp.dot(wdt, mt_ref[...],
                            preferred_element_type=jnp.float32) + bd_col
        maug_ref[:, pl.ds(0, C)] = m_t
        maug_ref[:, pl.ds(C, 8)] = jnp.broadcast_to((c_col - const_col) * 0.125,
                                                    (C, 8))
        const_ref[...] = jnp.broadcast_to(const_col, (C, 8))

    nfull = n_ref[...]
    li = jax.lax.broadcasted_iota(jnp.int32, (L, 1), 0)
    base = pl.program_id(0) * LB
    maug = maug_ref[...]
    const_col = const_ref[:, 0:1]

    for j in range(LB):
        nl = n_ref[pl.ds(base + j, 1), :]
        pred = (nfull < nl) | ((nfull == nl) & (li < base + j))
        rank = jnp.sum(pred.astype(jnp.float32), axis=0, keepdims=True)
        keep = jnp.where(rank < KEEP, 1.0, 0.0)
        mask_ref[j] = 1.0 - keep

        x_aug = jnp.concatenate(
            [x_ref[j] * keep, jnp.broadcast_to(keep, (8, B))], axis=0)
        dec = jnp.dot(maug, x_aug, preferred_element_type=jnp.float32)
        dec_ref[j] = dec + const_col


def kernel(x, noise, W_enc, b_enc, W_dec, b_dec, mask_token):
    B, C, H, W = x.shape
    L = H * W
    d_enc = W_enc.shape[1]
    x_t = x.transpose(2, 3, 1, 0).reshape(L, C, B)
    n_t = noise.T

    LB = 14
    grid = (L // LB,)

    dec_t, mask_t = pl.pallas_call(
        _mae_body,
        grid=grid,
        in_specs=[
            pl.BlockSpec((LB, C, B), lambda i: (i, 0, 0)),
            pl.BlockSpec((L, B), lambda i: (0, 0)),
            pl.BlockSpec((C, d_enc), lambda i: (0, 0)),
            pl.BlockSpec((C, d_enc), lambda i: (0, 0)),
            pl.BlockSpec((d_enc, 1), lambda i: (0, 0)),
            pl.BlockSpec((C, 1), lambda i: (0, 0)),
            pl.BlockSpec((d_enc, 1), lambda i: (0, 0)),
        ],
        out_specs=[
            pl.BlockSpec((LB, C, B), lambda i: (i, 0, 0)),
            pl.BlockSpec((LB, 1, B), lambda i: (i, 0, 0)),
        ],
        out_shape=[
            jax.ShapeDtypeStruct((L, C, B), jnp.float32),
            jax.ShapeDtypeStruct((L, 1, B), jnp.float32),
        ],
        scratch_shapes=[
            pltpu.VMEM((C, C + 8), jnp.float32),
            pltpu.VMEM((C, 8), jnp.float32),
        ],
    )(x_t, n_t, W_enc, W_dec.T, b_enc.reshape(d_enc, 1),
      b_dec.reshape(C, 1), mask_token.reshape(d_enc, 1))
    return dec_t.transpose(2, 0, 1), mask_t.reshape(L, B).T


# speedup vs baseline: 1068.2849x; 1.0441x over previous
import jax
import jax.numpy as jnp
from jax.experimental import pallas as pl
from jax.experimental.pallas import tpu as pltpu

_MASK_RATIO = 0.75


def _mae_body(x_ref, n_ref, we_ref, wdt_ref, be_ref, bd_ref, mt_ref,
              dec_ref, mask_ref, maug_ref, const_ref):
    LB, C, B = x_ref.shape
    L = n_ref.shape[0]
    KEEP = int(L * (1.0 - _MASK_RATIO))

    @pl.when(pl.program_id(0) == 0)
    def _init():
        wdt = wdt_ref[...]
        bd_col = bd_ref[...]
        m_t = jnp.dot(wdt, jnp.transpose(we_ref[...]),
                      preferred_element_type=jnp.float32)
        c_col = jnp.dot(wdt, be_ref[...],
                        preferred_element_type=jnp.float32) + bd_col
        const_col = jnp.dot(wdt, mt_ref[...],
                            preferred_element_type=jnp.float32) + bd_col
        maug_ref[:, pl.ds(0, C)] = m_t
        maug_ref[:, pl.ds(C, 8)] = jnp.broadcast_to((c_col - const_col) * 0.125,
                                                    (C, 8))
        const_ref[...] = jnp.broadcast_to(const_col, (C, 8))

    nfull = n_ref[...]
    li = jax.lax.broadcasted_iota(jnp.int32, (L, 1), 0)
    base = pl.program_id(0) * LB
    maug = maug_ref[...]
    const_col = const_ref[:, 0:1]

    for j in range(LB):
        nl = n_ref[pl.ds(base + j, 1), :]
        pred = (nfull < nl) | ((nfull == nl) & (li < base + j))
        rank = jnp.sum(pred.astype(jnp.float32), axis=0, keepdims=True)
        keep = jnp.where(rank < KEEP, 1.0, 0.0)
        mask_ref[pl.ds(base + j, 1), :] = 1.0 - keep

        x_aug = jnp.concatenate(
            [x_ref[j] * keep, jnp.broadcast_to(keep, (8, B))], axis=0)
        dec = jnp.dot(maug, x_aug, preferred_element_type=jnp.float32)
        dec_ref[j] = dec + const_col


def kernel(x, noise, W_enc, b_enc, W_dec, b_dec, mask_token):
    B, C, H, W = x.shape
    L = H * W
    d_enc = W_enc.shape[1]
    x_t = x.transpose(2, 3, 1, 0).reshape(L, C, B)
    n_t = noise.T

    LB = 14
    grid = (L // LB,)

    dec_t, mask_t = pl.pallas_call(
        _mae_body,
        grid=grid,
        in_specs=[
            pl.BlockSpec((LB, C, B), lambda i: (i, 0, 0)),
            pl.BlockSpec((L, B), lambda i: (0, 0)),
            pl.BlockSpec((C, d_enc), lambda i: (0, 0)),
            pl.BlockSpec((C, d_enc), lambda i: (0, 0)),
            pl.BlockSpec((d_enc, 1), lambda i: (0, 0)),
            pl.BlockSpec((C, 1), lambda i: (0, 0)),
            pl.BlockSpec((d_enc, 1), lambda i: (0, 0)),
        ],
        out_specs=[
            pl.BlockSpec((LB, C, B), lambda i: (i, 0, 0)),
            pl.BlockSpec((L, B), lambda i: (0, 0)),
        ],
        out_shape=[
            jax.ShapeDtypeStruct((L, C, B), jnp.float32),
            jax.ShapeDtypeStruct((L, B), jnp.float32),
        ],
        scratch_shapes=[
            pltpu.VMEM((C, C + 8), jnp.float32),
            pltpu.VMEM((C, 8), jnp.float32),
        ],
    )(x_t, n_t, W_enc, W_dec.T, b_enc.reshape(d_enc, 1),
      b_dec.reshape(C, 1), mask_token.reshape(d_enc, 1))
    return dec_t.transpose(2, 0, 1), mask_t.T


# speedup vs baseline: 1141.7391x; 1.0688x over previous
import jax
import jax.numpy as jnp
from jax.experimental import pallas as pl
from jax.experimental.pallas import tpu as pltpu

_MASK_RATIO = 0.75


def _mae_body(x_ref, n_ref, we_ref, wdt_ref, be_ref, bd_ref, mt_ref,
              dec_ref, mask_ref, maug_ref, const_ref):
    LB, C, B = x_ref.shape
    L = n_ref.shape[0]
    KEEP = int(L * (1.0 - _MASK_RATIO))

    @pl.when(pl.program_id(0) == 0)
    def _init():
        wdt = wdt_ref[...]
        bd_col = bd_ref[...]
        m_t = jnp.dot(wdt, jnp.transpose(we_ref[...]),
                      preferred_element_type=jnp.float32)
        c_col = jnp.dot(wdt, be_ref[...],
                        preferred_element_type=jnp.float32) + bd_col
        const_col = jnp.dot(wdt, mt_ref[...],
                            preferred_element_type=jnp.float32) + bd_col
        maug_ref[:, pl.ds(0, C)] = m_t
        maug_ref[:, pl.ds(C, 8)] = jnp.broadcast_to((c_col - const_col) * 0.125,
                                                    (C, 8))
        const_ref[...] = jnp.broadcast_to(const_col, (C, 8))

    nfull = n_ref[...]
    li = jax.lax.broadcasted_iota(jnp.int32, (L, 1), 0)
    base = pl.program_id(0) * LB
    maug = maug_ref[...]
    const_col = const_ref[:, 0:1]

    for j in range(LB):
        nl = n_ref[pl.ds(base + j, 1), :]
        pred = (nfull < nl) | ((nfull == nl) & (li < base + j))
        rank = jnp.sum(pred.astype(jnp.float32), axis=0, keepdims=True)
        keep = jnp.where(rank < KEEP, 1.0, 0.0)
        mask_ref[pl.ds(base + j, 1), :] = 1.0 - keep

        x_aug = jnp.concatenate(
            [x_ref[j] * keep, jnp.broadcast_to(keep, (8, B))], axis=0)
        dec = jnp.dot(maug, x_aug, preferred_element_type=jnp.float32)
        dec_ref[j] = dec + const_col


def kernel(x, noise, W_enc, b_enc, W_dec, b_dec, mask_token):
    B, C, H, W = x.shape
    L = H * W
    d_enc = W_enc.shape[1]
    x_t = x.transpose(2, 3, 1, 0).reshape(L, C, B)
    n_t = noise.T

    LB = 28
    grid = (L // LB,)

    dec_t, mask_t = pl.pallas_call(
        _mae_body,
        grid=grid,
        in_specs=[
            pl.BlockSpec((LB, C, B), lambda i: (i, 0, 0)),
            pl.BlockSpec((L, B), lambda i: (0, 0)),
            pl.BlockSpec((C, d_enc), lambda i: (0, 0)),
            pl.BlockSpec((C, d_enc), lambda i: (0, 0)),
            pl.BlockSpec((d_enc, 1), lambda i: (0, 0)),
            pl.BlockSpec((C, 1), lambda i: (0, 0)),
            pl.BlockSpec((d_enc, 1), lambda i: (0, 0)),
        ],
        out_specs=[
            pl.BlockSpec((LB, C, B), lambda i: (i, 0, 0)),
            pl.BlockSpec((L, B), lambda i: (0, 0)),
        ],
        out_shape=[
            jax.ShapeDtypeStruct((L, C, B), jnp.float32),
            jax.ShapeDtypeStruct((L, B), jnp.float32),
        ],
        scratch_shapes=[
            pltpu.VMEM((C, C + 8), jnp.float32),
            pltpu.VMEM((C, 8), jnp.float32),
        ],
    )(x_t, n_t, W_enc, W_dec.T, b_enc.reshape(d_enc, 1),
      b_dec.reshape(C, 1), mask_token.reshape(d_enc, 1))
    return dec_t.transpose(2, 0, 1), mask_t.T


# speedup vs baseline: 1159.4208x; 1.0155x over previous
import jax
import jax.numpy as jnp
from jax.experimental import pallas as pl
from jax.experimental.pallas import tpu as pltpu

_MASK_RATIO = 0.75


def _mae_body(x_ref, n_ref, we_ref, wdt_ref, be_ref, bd_ref, mt_ref,
              dec_ref, mask_ref, maug_ref, const_ref):
    LB, C, B = x_ref.shape
    L = n_ref.shape[0]
    KEEP = int(L * (1.0 - _MASK_RATIO))

    @pl.when(pl.program_id(0) == 0)
    def _init():
        wdt = wdt_ref[...]
        bd_col = bd_ref[...]
        m_t = jnp.dot(wdt, jnp.transpose(we_ref[...]),
                      preferred_element_type=jnp.float32)
        c_col = jnp.dot(wdt, be_ref[...],
                        preferred_element_type=jnp.float32) + bd_col
        const_col = jnp.dot(wdt, mt_ref[...],
                            preferred_element_type=jnp.float32) + bd_col
        maug_ref[:, pl.ds(0, C)] = m_t
        maug_ref[:, pl.ds(C, 8)] = jnp.broadcast_to((c_col - const_col) * 0.125,
                                                    (C, 8))
        const_ref[...] = jnp.broadcast_to(const_col, (C, 8))

    nfull = n_ref[...]
    li = jax.lax.broadcasted_iota(jnp.int32, (L, 1), 0)
    base = pl.program_id(0) * LB
    maug = maug_ref[...]
    const_col = const_ref[:, 0:1]

    for j in range(LB):
        nl = n_ref[pl.ds(base + j, 1), :]
        pred = (nfull < nl) | ((nfull == nl) & (li < base + j))
        rank = jnp.sum(pred.astype(jnp.float32), axis=0, keepdims=True)
        keep = jnp.where(rank < KEEP, 1.0, 0.0)
        mask_ref[pl.ds(base + j, 1), :] = 1.0 - keep

        x_aug = jnp.concatenate(
            [x_ref[j] * keep, jnp.broadcast_to(keep, (8, B))], axis=0)
        dec = jnp.dot(maug, x_aug, preferred_element_type=jnp.float32)
        dec_ref[j] = dec + const_col


def kernel(x, noise, W_enc, b_enc, W_dec, b_dec, mask_token):
    B, C, H, W = x.shape
    L = H * W
    d_enc = W_enc.shape[1]
    x_t = x.transpose(2, 3, 1, 0).reshape(L, C, B)
    n_t = noise.T

    LB = 49
    grid = (L // LB,)

    dec_t, mask_t = pl.pallas_call(
        _mae_body,
        grid=grid,
        in_specs=[
            pl.BlockSpec((LB, C, B), lambda i: (i, 0, 0)),
            pl.BlockSpec((L, B), lambda i: (0, 0)),
            pl.BlockSpec((C, d_enc), lambda i: (0, 0)),
            pl.BlockSpec((C, d_enc), lambda i: (0, 0)),
            pl.BlockSpec((d_enc, 1), lambda i: (0, 0)),
            pl.BlockSpec((C, 1), lambda i: (0, 0)),
            pl.BlockSpec((d_enc, 1), lambda i: (0, 0)),
        ],
        out_specs=[
            pl.BlockSpec((LB, C, B), lambda i: (i, 0, 0)),
            pl.BlockSpec((L, B), lambda i: (0, 0)),
        ],
        out_shape=[
            jax.ShapeDtypeStruct((L, C, B), jnp.float32),
            jax.ShapeDtypeStruct((L, B), jnp.float32),
        ],
        scratch_shapes=[
            pltpu.VMEM((C, C + 8), jnp.float32),
            pltpu.VMEM((C, 8), jnp.float32),
        ],
    )(x_t, n_t, W_enc, W_dec.T, b_enc.reshape(d_enc, 1),
      b_dec.reshape(C, 1), mask_token.reshape(d_enc, 1))
    return dec_t.transpose(2, 0, 1), mask_t.T


# speedup vs baseline: 1172.5047x; 1.0113x over previous
import jax
import jax.numpy as jnp
from jax.experimental import pallas as pl
from jax.experimental.pallas import tpu as pltpu

_MASK_RATIO = 0.75


def _mae_body(x_ref, n_ref, we_ref, wdt_ref, be_ref, bd_ref, mt_ref,
              dec_ref, mask_ref, maug_ref, const_ref):
    LB, C, B = x_ref.shape
    L = n_ref.shape[0]
    KEEP = int(L * (1.0 - _MASK_RATIO))

    @pl.when((pl.program_id(0) + pl.program_id(1)) == 0)
    def _init():
        wdt = wdt_ref[...]
        bd_col = bd_ref[...]
        m_t = jnp.dot(wdt, jnp.transpose(we_ref[...]),
                      preferred_element_type=jnp.float32)
        c_col = jnp.dot(wdt, be_ref[...],
                        preferred_element_type=jnp.float32) + bd_col
        const_col = jnp.dot(wdt, mt_ref[...],
                            preferred_element_type=jnp.float32) + bd_col
        maug_ref[:, pl.ds(0, C)] = m_t
        maug_ref[:, pl.ds(C, 8)] = jnp.broadcast_to((c_col - const_col) * 0.125,
                                                    (C, 8))
        const_ref[...] = jnp.broadcast_to(const_col, (C, 8))

    nfull = n_ref[...]
    li = jax.lax.broadcasted_iota(jnp.int32, (L, 1), 0)
    base = pl.program_id(1) * LB
    maug = maug_ref[...]
    const_col = const_ref[:, 0:1]

    for j in range(LB):
        nl = n_ref[pl.ds(base + j, 1), :]
        pred = (nfull < nl) | ((nfull == nl) & (li < base + j))
        rank = jnp.sum(pred.astype(jnp.float32), axis=0, keepdims=True)
        keep = jnp.where(rank < KEEP, 1.0, 0.0)
        mask_ref[pl.ds(base + j, 1), :] = 1.0 - keep

        x_aug = jnp.concatenate(
            [x_ref[j] * keep, jnp.broadcast_to(keep, (8, B))], axis=0)
        dec = jnp.dot(maug, x_aug, preferred_element_type=jnp.float32)
        dec_ref[j] = dec + const_col


def kernel(x, noise, W_enc, b_enc, W_dec, b_dec, mask_token):
    B, C, H, W = x.shape
    L = H * W
    d_enc = W_enc.shape[1]
    x_t = x.transpose(2, 3, 1, 0).reshape(L, C, B)
    n_t = noise.T

    LB = 49
    BB = 128
    grid = (B // BB, L // LB)

    dec_t, mask_t = pl.pallas_call(
        _mae_body,
        grid=grid,
        in_specs=[
            pl.BlockSpec((LB, C, BB), lambda h, i: (i, 0, h)),
            pl.BlockSpec((L, BB), lambda h, i: (0, h)),
            pl.BlockSpec((C, d_enc), lambda h, i: (0, 0)),
            pl.BlockSpec((C, d_enc), lambda h, i: (0, 0)),
            pl.BlockSpec((d_enc, 1), lambda h, i: (0, 0)),
            pl.BlockSpec((C, 1), lambda h, i: (0, 0)),
            pl.BlockSpec((d_enc, 1), lambda h, i: (0, 0)),
        ],
        out_specs=[
            pl.BlockSpec((LB, C, BB), lambda h, i: (i, 0, h)),
            pl.BlockSpec((L, BB), lambda h, i: (0, h)),
        ],
        out_shape=[
            jax.ShapeDtypeStruct((L, C, B), jnp.float32),
            jax.ShapeDtypeStruct((L, B), jnp.float32),
        ],
        scratch_shapes=[
            pltpu.VMEM((C, C + 8), jnp.float32),
            pltpu.VMEM((C, 8), jnp.float32),
        ],
    )(x_t, n_t, W_enc, W_dec.T, b_enc.reshape(d_enc, 1),
      b_dec.reshape(C, 1), mask_token.reshape(d_enc, 1))
    return dec_t.transpose(2, 0, 1), mask_t.T
